# Initial kernel scaffold; baseline (speedup 1.0000x reference)
import jax
import jax.numpy as jnp
from jax.experimental import pallas as pl


def kernel(noisy_data, data, condition_mask):
    raise NotImplementedError("write your pallas kernel here")


# speedup vs baseline: 28.1410x; 28.1410x over previous
import jax
import jax.numpy as jnp
from jax.experimental import pallas as pl


_B, _NS, _D = 8, 2048, 128
_BQ = 256


def _nn_kernel(x_ref, y_ref, md_ref, idx_ref):
    x = x_ref[0]
    y = y_ref[0]
    g = jax.lax.dot_general(
        x, y,
        dimension_numbers=(((1,), (1,)), ((), ())),
        precision=jax.lax.Precision.HIGHEST,
        preferred_element_type=jnp.float32,
    )
    xn = jnp.sum(x * x, axis=1, keepdims=True)
    yn = jnp.sum(y * y, axis=1, keepdims=True).T
    d2 = (xn - 2.0 * g) + yn
    m = jnp.min(d2, axis=1, keepdims=True)
    iota = jax.lax.broadcasted_iota(jnp.int32, d2.shape, 1)
    hit = jnp.where(d2 <= m, iota, jnp.int32(_NS))
    idx = jnp.min(hit, axis=1)
    md_ref[0, 0, :] = jnp.sqrt(jnp.maximum(m[:, 0], 0.0))
    idx_ref[0, 0, :] = idx


def kernel(noisy_data, data, condition_mask):
    x = jnp.where(condition_mask[None, None, :], data, noisy_data)
    nq = _NS // _BQ
    grid = (_B, nq)
    md, idx = pl.pallas_call(
        _nn_kernel,
        grid=grid,
        in_specs=[
            pl.BlockSpec((1, _BQ, _D), lambda b, q: (b, q, 0)),
            pl.BlockSpec((1, _NS, _D), lambda b, q: (b, 0, 0)),
        ],
        out_specs=[
            pl.BlockSpec((1, 1, _BQ), lambda b, q: (b * nq + q, 0, 0)),
            pl.BlockSpec((1, 1, _BQ), lambda b, q: (b * nq + q, 0, 0)),
        ],
        out_shape=[
            jax.ShapeDtypeStruct((_B * nq, 1, _BQ), jnp.float32),
            jax.ShapeDtypeStruct((_B * nq, 1, _BQ), jnp.int32),
        ],
    )(x, data)
    return md.reshape(_B, _NS), idx.reshape(_B, _NS)


# speedup vs baseline: 32.4525x; 1.1532x over previous
import jax
import jax.numpy as jnp
from jax.experimental import pallas as pl
from jax.experimental.pallas import tpu as pltpu


_B, _NS, _D = 8, 2048, 128
_BQ = 512


def _nn_kernel(x_ref, y_ref, md_ref, idx_ref, yn_ref):
    q = pl.program_id(1)
    y = y_ref[0]

    @pl.when(q == 0)
    def _fill_yn():
        yn_ref[...] = jnp.sum(y * y, axis=1, keepdims=True)

    x = x_ref[0]
    g = jax.lax.dot_general(
        y, x * -2.0,
        dimension_numbers=(((1,), (1,)), ((), ())),
        precision=jax.lax.Precision.HIGHEST,
        preferred_element_type=jnp.float32,
    )
    d2 = g + yn_ref[...]
    m = jnp.min(d2, axis=0, keepdims=True)
    iota = jax.lax.broadcasted_iota(jnp.int32, d2.shape, 0)
    hit = jnp.where(d2 <= m, iota, jnp.int32(_NS))
    idx = jnp.min(hit, axis=0)
    xn = jax.lax.dot_general(
        jnp.ones((1, _D), jnp.float32), x * x,
        dimension_numbers=(((1,), (1,)), ((), ())),
        precision=jax.lax.Precision.HIGHEST,
        preferred_element_type=jnp.float32,
    )
    md_ref[0, 0, :] = jnp.sqrt(jnp.maximum(xn[0] + m[0], 0.0))
    idx_ref[0, 0, :] = idx


def kernel(noisy_data, data, condition_mask):
    x = jnp.where(condition_mask[None, None, :], data, noisy_data)
    nq = _NS // _BQ
    grid = (_B, nq)
    md, idx = pl.pallas_call(
        _nn_kernel,
        grid=grid,
        in_specs=[
            pl.BlockSpec((1, _BQ, _D), lambda b, q: (b, q, 0)),
            pl.BlockSpec((1, _NS, _D), lambda b, q: (b, 0, 0)),
        ],
        out_specs=[
            pl.BlockSpec((1, 1, _BQ), lambda b, q: (b * nq + q, 0, 0)),
            pl.BlockSpec((1, 1, _BQ), lambda b, q: (b * nq + q, 0, 0)),
        ],
        out_shape=[
            jax.ShapeDtypeStruct((_B * nq, 1, _BQ), jnp.float32),
            jax.ShapeDtypeStruct((_B * nq, 1, _BQ), jnp.int32),
        ],
        scratch_shapes=[pltpu.VMEM((_NS, 1), jnp.float32)],
    )(x, data)
    return md.reshape(_B, _NS), idx.reshape(_B, _NS)
